# baseline (speedup 1.0000x reference)
import jax
import jax.numpy as jnp
from jax import lax
from jax.experimental import pallas as pl
from jax.experimental.pallas import tpu as pltpu, tpu_sc as plsc

N_NODES = 10000
N_EDGES = 320000
DIM = 128
COUT = 16

NC = 2
NS = 16
NW = NC * NS
ROWS_PER_TILE = 640
N_PAD = NS * ROWS_PER_TILE
EDGE_CHUNK = 128
CHUNKS_PER_WORKER = 80
E_PAD = NW * CHUNKS_PER_WORKER * EDGE_CHUNK
RING = 8
DEPTH = 4

TAB = 327808
TAB_GRID = 8
TAB_BLK = TAB // TAB_GRID

MM_GRID = 8
MM_ROWS = N_PAD // MM_GRID


def _mm_body(x_ref, w_ref, o_ref):
    o_ref[...] = jnp.dot(x_ref[...], w_ref[...],
                         preferred_element_type=jnp.float32)


def _matmul(x_pad, W):
    return pl.pallas_call(
        _mm_body,
        grid=(MM_GRID,),
        in_specs=[
            pl.BlockSpec((MM_ROWS, DIM), lambda i: (i, 0)),
            pl.BlockSpec((DIM, COUT), lambda i: (0, 0)),
        ],
        out_specs=pl.BlockSpec((MM_ROWS, COUT), lambda i: (i, 0)),
        out_shape=jax.ShapeDtypeStruct((N_PAD, COUT), jnp.float32),
    )(x_pad, W)


def _tab_body(o_ref):
    i = pl.program_id(0)
    k = (lax.broadcasted_iota(jnp.int32, (TAB_BLK, COUT), 0)
         + i * TAB_BLK)
    x = k.astype(jnp.float32) + 1.0
    y = lax.rsqrt(x)
    o_ref[...] = y * (1.5 - 0.5 * x * y * y)


def _rsqrt_table():
    return pl.pallas_call(
        _tab_body,
        grid=(TAB_GRID,),
        out_specs=pl.BlockSpec((TAB_BLK, COUT), lambda i: (i, 0)),
        out_shape=jax.ShapeDtypeStruct((TAB, COUT), jnp.float32),
    )()


def _softmax_body(acc_ref, deg_ref, b_ref, o_ref):
    i = pl.program_id(0)
    deg = deg_ref[pl.ds(i * MM_ROWS, MM_ROWS)]
    dinv = lax.rsqrt(deg + 1.0)[:, None]
    v = (acc_ref[0] + acc_ref[1]) * dinv + b_ref[...]
    m = jnp.max(v, axis=1, keepdims=True)
    e = jnp.exp(v - m)
    o_ref[...] = e / jnp.sum(e, axis=1, keepdims=True)


def _softmax(acc, deg, b):
    return pl.pallas_call(
        _softmax_body,
        grid=(MM_GRID,),
        in_specs=[
            pl.BlockSpec((NC, MM_ROWS, COUT), lambda i: (0, i, 0)),
            pl.BlockSpec((N_PAD,), lambda i: (0,)),
            pl.BlockSpec((COUT,), lambda i: (0,)),
        ],
        out_specs=pl.BlockSpec((MM_ROWS, COUT), lambda i: (i, 0)),
        out_shape=jax.ShapeDtypeStruct((N_PAD, COUT), jnp.float32),
    )(acc, deg, b)


def _sc_body(xw_hbm, tab_hbm, src_hbm, dst_hbm, out_hbm, deg_hbm,
             deg_s, ywt_s, acc_s,
             src_v, dst_v, dst2_v, ones_v, rows_v, a_v, z1_v, z2_v,
             d_v, idx_v, drows_v, gsem, ssem):
    c = lax.axis_index("c")
    s = lax.axis_index("s")
    w = c * NS + s
    base = s * ROWS_PER_TILE

    def _z1(i, cy):
        z1_v[pl.ds(i * COUT, COUT)] = jnp.zeros((COUT,), jnp.float32)
        return cy
    lax.fori_loop(0, ROWS_PER_TILE // COUT, _z1, 0)

    def _z2(i, cy):
        z2_v[i, :] = jnp.zeros((COUT,), jnp.float32)
        return cy
    lax.fori_loop(0, ROWS_PER_TILE, _z2, 0)

    def _ones(i, cy):
        ones_v[pl.ds(i * COUT, COUT)] = jnp.ones((COUT,), jnp.float32)
        return cy
    lax.fori_loop(0, EDGE_CHUNK // COUT, _ones, 0)

    pltpu.sync_copy(z1_v, deg_s.at[pl.ds(base, ROWS_PER_TILE)])
    pltpu.sync_copy(src_hbm.at[w], src_v)
    pltpu.sync_copy(dst_hbm.at[w], dst_v)
    w2 = (1 - c) * NS + s
    pltpu.sync_copy(dst_hbm.at[w2], dst2_v)
    plsc.subcore_barrier()

    def _deg(j, cy):
        pltpu.async_copy(ones_v, deg_s.at[dst_v.at[j]], gsem, add=True)
        pltpu.async_copy(ones_v, deg_s.at[dst2_v.at[j]], gsem, add=True)
        return cy
    lax.fori_loop(0, CHUNKS_PER_WORKER, _deg, 0)

    def _degdrain(j, cy):
        pltpu.make_async_copy(ones_v, deg_s.at[dst_v.at[0]], gsem).wait()
        return cy
    lax.fori_loop(0, 2 * CHUNKS_PER_WORKER, _degdrain, 0)
    plsc.subcore_barrier()

    pltpu.sync_copy(deg_s.at[pl.ds(base, ROWS_PER_TILE)], d_v)

    def _dg(r, cy):
        def _conv(k, c2):
            vals = d_v[pl.ds(r * 128 + k * COUT, COUT)]
            idx_v[pl.ds(k * COUT, COUT)] = vals.astype(jnp.int32)
            return c2
        lax.fori_loop(0, 128 // COUT, _conv, 0)
        pltpu.sync_copy(tab_hbm.at[idx_v],
                        drows_v.at[pl.ds(r * 128, 128)])
        return cy
    lax.fori_loop(0, ROWS_PER_TILE // 128, _dg, 0)
    pltpu.sync_copy(xw_hbm.at[pl.ds(base, ROWS_PER_TILE)], a_v)

    def _scale(i, cy):
        a_v[i, :] = a_v[i, :] * drows_v[i, :]
        return cy
    lax.fori_loop(0, ROWS_PER_TILE, _scale, 0)

    pltpu.sync_copy(a_v, ywt_s.at[pl.ds(base, ROWS_PER_TILE)])

    @pl.when(c == 0)
    def _():
        pltpu.sync_copy(a_v, acc_s.at[pl.ds(base, ROWS_PER_TILE)])

    @pl.when(c != 0)
    def _():
        pltpu.sync_copy(z2_v, acc_s.at[pl.ds(base, ROWS_PER_TILE)])

    @pl.when(c == 0)
    def _():
        pltpu.sync_copy(d_v, deg_hbm.at[pl.ds(base, ROWS_PER_TILE)])

    plsc.subcore_barrier()

    pltpu.async_copy(ywt_s.at[src_v.at[0]], rows_v.at[0], gsem)

    def _msg(j, cy):
        b = j % 2
        nb = (j + 1) % 2
        pltpu.make_async_copy(ywt_s.at[src_v.at[j]], rows_v.at[b],
                              gsem).wait()

        @pl.when(j >= 1)
        def _():
            pltpu.make_async_copy(rows_v.at[nb], acc_s.at[dst_v.at[0]],
                                  ssem).wait()

        @pl.when(j + 1 < CHUNKS_PER_WORKER)
        def _():
            pltpu.async_copy(ywt_s.at[src_v.at[j + 1]], rows_v.at[nb], gsem)

        pltpu.async_copy(rows_v.at[b], acc_s.at[dst_v.at[j]], ssem, add=True)
        return cy
    lax.fori_loop(0, CHUNKS_PER_WORKER, _msg, 0)
    pltpu.make_async_copy(rows_v.at[0], acc_s.at[dst_v.at[0]], ssem).wait()
    plsc.subcore_barrier()

    pltpu.sync_copy(acc_s.at[pl.ds(base, ROWS_PER_TILE)], a_v)
    pltpu.sync_copy(a_v, out_hbm.at[c, pl.ds(base, ROWS_PER_TILE)])


@jax.jit
def kernel(x, edge_index, W, b):
    n = x.shape[0]
    x_pad = jnp.zeros((N_PAD, DIM), jnp.float32).at[:n].set(x)
    xw = _matmul(x_pad, W)
    tab = _rsqrt_table()

    src = jnp.full((E_PAD,), n, jnp.int32).at[:N_EDGES].set(edge_index[0])
    dst = jnp.full((E_PAD,), n, jnp.int32).at[:N_EDGES].set(edge_index[1])
    src = src.reshape(NW, CHUNKS_PER_WORKER, EDGE_CHUNK)
    dst = dst.reshape(NW, CHUNKS_PER_WORKER, EDGE_CHUNK)

    mesh = plsc.VectorSubcoreMesh(core_axis_name="c", subcore_axis_name="s",
                                  num_cores=NC)
    sc_params = pltpu.CompilerParams(use_tc_tiling_on_sc=False)

    sc_kernel = pl.kernel(
        _sc_body,
        out_type=(
            jax.ShapeDtypeStruct((NC, N_PAD, COUT), jnp.float32),
            jax.ShapeDtypeStruct((N_PAD,), jnp.float32),
        ),
        mesh=mesh,
        compiler_params=sc_params,
        scratch_types=[
            pltpu.VMEM_SHARED((N_PAD,), jnp.float32),
            pltpu.VMEM_SHARED((N_PAD, COUT), jnp.float32),
            pltpu.VMEM_SHARED((N_PAD, COUT), jnp.float32),
            pltpu.VMEM((CHUNKS_PER_WORKER, EDGE_CHUNK), jnp.int32),
            pltpu.VMEM((CHUNKS_PER_WORKER, EDGE_CHUNK), jnp.int32),
            pltpu.VMEM((CHUNKS_PER_WORKER, EDGE_CHUNK), jnp.int32),
            pltpu.VMEM((EDGE_CHUNK,), jnp.float32),
            pltpu.VMEM((RING, EDGE_CHUNK, COUT), jnp.float32),
            pltpu.VMEM((ROWS_PER_TILE, COUT), jnp.float32),
            pltpu.VMEM((ROWS_PER_TILE,), jnp.float32),
            pltpu.VMEM((ROWS_PER_TILE, COUT), jnp.float32),
            pltpu.VMEM((ROWS_PER_TILE,), jnp.float32),
            pltpu.VMEM((EDGE_CHUNK,), jnp.int32),
            pltpu.VMEM((ROWS_PER_TILE, COUT), jnp.float32),
            pltpu.SemaphoreType.DMA,
            pltpu.SemaphoreType.DMA,
        ],
    )
    acc, deg = sc_kernel(xw, tab, src, dst)
    out = _softmax(acc, deg, b)
    return out[:n]


# speedup vs baseline: 2.9409x; 2.9409x over previous
import jax
import jax.numpy as jnp
from jax import lax
from jax.experimental import pallas as pl
from jax.experimental.pallas import tpu as pltpu, tpu_sc as plsc

N_NODES = 10000
N_EDGES = 320000
DIM = 128
COUT = 16

NC = 2
NS = 16
NW = NC * NS
ROWS_PER_TILE = 640
N_PAD = NS * ROWS_PER_TILE
EDGE_CHUNK = 128
CHUNKS_PER_WORKER = 79
E_PAD = NW * CHUNKS_PER_WORKER * EDGE_CHUNK
RING = 8
DEPTH = 4

MM_GRID = 8
MM_ROWS = N_PAD // MM_GRID


def _mm_body(x_ref, w_ref, o_ref):
    o_ref[...] = jnp.dot(x_ref[...], w_ref[...],
                         preferred_element_type=jnp.float32)


def _matmul(x_pad, W):
    return pl.pallas_call(
        _mm_body,
        grid=(MM_GRID,),
        in_specs=[
            pl.BlockSpec((MM_ROWS, DIM), lambda i: (i, 0)),
            pl.BlockSpec((DIM, COUT), lambda i: (0, 0)),
        ],
        out_specs=pl.BlockSpec((MM_ROWS, COUT), lambda i: (i, 0)),
        out_shape=jax.ShapeDtypeStruct((N_PAD, COUT), jnp.float32),
    )(x_pad, W)


def _norm_body(xw_ref, deg_ref, yw_ref, dinv_ref):
    i = pl.program_id(0)
    deg = (deg_ref[0, pl.ds(i * MM_ROWS, MM_ROWS)]
           + deg_ref[1, pl.ds(i * MM_ROWS, MM_ROWS)])
    dinv = lax.rsqrt(deg + 1.0)[:, None]
    dinv_ref[...] = jnp.broadcast_to(dinv, (MM_ROWS, COUT))
    yw_ref[...] = xw_ref[...] * dinv


def _normalize(xw, deg):
    return pl.pallas_call(
        _norm_body,
        grid=(MM_GRID,),
        in_specs=[
            pl.BlockSpec((MM_ROWS, COUT), lambda i: (i, 0)),
            pl.BlockSpec((NC, N_PAD), lambda i: (0, 0)),
        ],
        out_specs=[
            pl.BlockSpec((MM_ROWS, COUT), lambda i: (i, 0)),
            pl.BlockSpec((MM_ROWS, COUT), lambda i: (i, 0)),
        ],
        out_shape=[
            jax.ShapeDtypeStruct((N_PAD, COUT), jnp.float32),
            jax.ShapeDtypeStruct((N_PAD, COUT), jnp.float32),
        ],
    )(xw, deg)


def _softmax_body(acc_ref, dinv_ref, b_ref, o_ref):
    v = (acc_ref[0] + acc_ref[1]) * dinv_ref[...] + b_ref[...]
    m = jnp.max(v, axis=1, keepdims=True)
    e = jnp.exp(v - m)
    o_ref[...] = e / jnp.sum(e, axis=1, keepdims=True)


def _softmax(acc, dinv, b):
    return pl.pallas_call(
        _softmax_body,
        grid=(MM_GRID,),
        in_specs=[
            pl.BlockSpec((NC, MM_ROWS, COUT), lambda i: (0, i, 0)),
            pl.BlockSpec((MM_ROWS, COUT), lambda i: (i, 0)),
            pl.BlockSpec((COUT,), lambda i: (0,)),
        ],
        out_specs=pl.BlockSpec((MM_ROWS, COUT), lambda i: (i, 0)),
        out_shape=jax.ShapeDtypeStruct((N_PAD, COUT), jnp.float32),
    )(acc, dinv, b)


def _sc_deg_body(dst_hbm, deg_hbm, deg_s, dst_v, ones_v, z_v, sem):
    c = lax.axis_index("c")
    s = lax.axis_index("s")
    w = c * NS + s
    base = s * ROWS_PER_TILE

    def _zero(i, cy):
        z_v[pl.ds(i * COUT, COUT)] = jnp.zeros((COUT,), jnp.float32)
        return cy
    lax.fori_loop(0, ROWS_PER_TILE // COUT, _zero, 0)

    def _ones(i, cy):
        ones_v[pl.ds(i * COUT, COUT)] = jnp.ones((COUT,), jnp.float32)
        return cy
    lax.fori_loop(0, EDGE_CHUNK // COUT, _ones, 0)

    pltpu.sync_copy(z_v, deg_s.at[pl.ds(base, ROWS_PER_TILE)])
    pltpu.sync_copy(dst_hbm.at[w], dst_v)
    plsc.subcore_barrier()

    def _deg(j, cy):
        pltpu.async_copy(ones_v, deg_s.at[dst_v.at[j]], sem, add=True)
        return cy
    lax.fori_loop(0, CHUNKS_PER_WORKER, _deg, 0)

    def _drain(j, cy):
        pltpu.make_async_copy(ones_v, deg_s.at[dst_v.at[0]], sem).wait()
        return cy
    lax.fori_loop(0, CHUNKS_PER_WORKER, _drain, 0)
    plsc.subcore_barrier()

    pltpu.sync_copy(deg_s.at[pl.ds(base, ROWS_PER_TILE)], z_v)
    pltpu.sync_copy(z_v, deg_hbm.at[c, pl.ds(base, ROWS_PER_TILE)])


def _sc_msg_body(yw_hbm, src_hbm, dst_hbm, out_hbm,
                 ywt_s, acc_s, src_v, dst_v, rows_v, a_v, z_v, gsem, ssem):
    c = lax.axis_index("c")
    s = lax.axis_index("s")
    w = c * NS + s
    base = s * ROWS_PER_TILE

    pltpu.sync_copy(yw_hbm.at[pl.ds(base, ROWS_PER_TILE)], a_v)
    pltpu.sync_copy(a_v, ywt_s.at[pl.ds(base, ROWS_PER_TILE)])

    def _zero(i, cy):
        z_v[i, :] = jnp.zeros((COUT,), jnp.float32)
        return cy
    lax.fori_loop(0, ROWS_PER_TILE, _zero, 0)

    @pl.when(c == 0)
    def _():
        pltpu.sync_copy(a_v, acc_s.at[pl.ds(base, ROWS_PER_TILE)])

    @pl.when(c != 0)
    def _():
        pltpu.sync_copy(z_v, acc_s.at[pl.ds(base, ROWS_PER_TILE)])

    pltpu.sync_copy(src_hbm.at[w], src_v)
    pltpu.sync_copy(dst_hbm.at[w], dst_v)
    plsc.subcore_barrier()

    for p in range(DEPTH):
        pltpu.async_copy(ywt_s.at[src_v.at[p]], rows_v.at[p], gsem)

    def _msg(j, cy):
        pltpu.make_async_copy(ywt_s.at[src_v.at[j]], rows_v.at[j % RING],
                              gsem).wait()
        pltpu.async_copy(rows_v.at[j % RING], acc_s.at[dst_v.at[j]],
                         ssem, add=True)

        @pl.when(j >= DEPTH - 1)
        def _():
            pltpu.make_async_copy(rows_v.at[0], acc_s.at[dst_v.at[0]],
                                  ssem).wait()

        @pl.when(j + DEPTH < CHUNKS_PER_WORKER)
        def _():
            pltpu.async_copy(ywt_s.at[src_v.at[j + DEPTH]],
                             rows_v.at[(j + DEPTH) % RING], gsem)
        return cy
    lax.fori_loop(0, CHUNKS_PER_WORKER, _msg, 0)

    def _sdrain(j, cy):
        pltpu.make_async_copy(rows_v.at[0], acc_s.at[dst_v.at[0]],
                              ssem).wait()
        return cy
    lax.fori_loop(0, DEPTH - 1, _sdrain, 0)
    plsc.subcore_barrier()

    pltpu.sync_copy(acc_s.at[pl.ds(base, ROWS_PER_TILE)], a_v)
    pltpu.sync_copy(a_v, out_hbm.at[c, pl.ds(base, ROWS_PER_TILE)])


@jax.jit
def kernel(x, edge_index, W, b):
    n = x.shape[0]
    x_pad = jnp.zeros((N_PAD, DIM), jnp.float32).at[:n].set(x)
    xw = _matmul(x_pad, W)

    src = jnp.full((E_PAD,), n, jnp.int32).at[:N_EDGES].set(edge_index[0])
    dst = jnp.full((E_PAD,), n, jnp.int32).at[:N_EDGES].set(edge_index[1])
    src = src.reshape(NW, CHUNKS_PER_WORKER, EDGE_CHUNK)
    dst = dst.reshape(NW, CHUNKS_PER_WORKER, EDGE_CHUNK)

    mesh = plsc.VectorSubcoreMesh(core_axis_name="c", subcore_axis_name="s",
                                  num_cores=NC)
    sc_params = pltpu.CompilerParams(use_tc_tiling_on_sc=False)

    deg_kernel = pl.kernel(
        _sc_deg_body,
        out_type=jax.ShapeDtypeStruct((NC, N_PAD), jnp.float32),
        mesh=mesh,
        compiler_params=sc_params,
        scratch_types=[
            pltpu.VMEM_SHARED((N_PAD,), jnp.float32),
            pltpu.VMEM((CHUNKS_PER_WORKER, EDGE_CHUNK), jnp.int32),
            pltpu.VMEM((EDGE_CHUNK,), jnp.float32),
            pltpu.VMEM((ROWS_PER_TILE,), jnp.float32),
            pltpu.SemaphoreType.DMA,
        ],
    )
    deg = deg_kernel(dst)

    yw, dinv = _normalize(xw, deg)

    msg_kernel = pl.kernel(
        _sc_msg_body,
        out_type=jax.ShapeDtypeStruct((NC, N_PAD, COUT), jnp.float32),
        mesh=mesh,
        compiler_params=sc_params,
        scratch_types=[
            pltpu.VMEM_SHARED((N_PAD, COUT), jnp.float32),
            pltpu.VMEM_SHARED((N_PAD, COUT), jnp.float32),
            pltpu.VMEM((CHUNKS_PER_WORKER, EDGE_CHUNK), jnp.int32),
            pltpu.VMEM((CHUNKS_PER_WORKER, EDGE_CHUNK), jnp.int32),
            pltpu.VMEM((RING, EDGE_CHUNK, COUT), jnp.float32),
            pltpu.VMEM((ROWS_PER_TILE, COUT), jnp.float32),
            pltpu.VMEM((ROWS_PER_TILE, COUT), jnp.float32),
            pltpu.SemaphoreType.DMA,
            pltpu.SemaphoreType.DMA,
        ],
    )
    acc = msg_kernel(yw, src, dst)
    out = _softmax(acc, dinv, b)
    return out[:n]


# speedup vs baseline: 3.0319x; 1.0310x over previous
import jax
import jax.numpy as jnp
from jax import lax
from jax.experimental import pallas as pl
from jax.experimental.pallas import tpu as pltpu, tpu_sc as plsc

N_NODES = 10000
N_EDGES = 320000
DIM = 128
COUT = 16

NC = 2
NS = 16
NW = NC * NS
ROWS_PER_TILE = 640
N_PAD = NS * ROWS_PER_TILE
EDGE_CHUNK = 128
CHUNKS_PER_WORKER = 79
E_PAD = NW * CHUNKS_PER_WORKER * EDGE_CHUNK
RING = 8
DEPTH = 4

MM_GRID = 8
MM_ROWS = N_PAD // MM_GRID


def _mmn_body(x_ref, w_ref, deg_ref, yw_ref, dinv_ref):
    i = pl.program_id(0)
    xw = jnp.dot(x_ref[...], w_ref[...],
                 preferred_element_type=jnp.float32)
    deg = (deg_ref[0, pl.ds(i * MM_ROWS, MM_ROWS)]
           + deg_ref[1, pl.ds(i * MM_ROWS, MM_ROWS)])
    dinv = lax.rsqrt(deg + 1.0)[:, None]
    dinv_ref[...] = jnp.broadcast_to(dinv, (MM_ROWS, COUT))
    yw_ref[...] = xw * dinv


def _matmul_normalize(x_pad, W, deg):
    return pl.pallas_call(
        _mmn_body,
        grid=(MM_GRID,),
        in_specs=[
            pl.BlockSpec((MM_ROWS, DIM), lambda i: (i, 0)),
            pl.BlockSpec((DIM, COUT), lambda i: (0, 0)),
            pl.BlockSpec((NC, N_PAD), lambda i: (0, 0)),
        ],
        out_specs=[
            pl.BlockSpec((MM_ROWS, COUT), lambda i: (i, 0)),
            pl.BlockSpec((MM_ROWS, COUT), lambda i: (i, 0)),
        ],
        out_shape=[
            jax.ShapeDtypeStruct((N_PAD, COUT), jnp.float32),
            jax.ShapeDtypeStruct((N_PAD, COUT), jnp.float32),
        ],
    )(x_pad, W, deg)


def _softmax_body(acc_ref, dinv_ref, b_ref, o_ref):
    v = (acc_ref[0] + acc_ref[1]) * dinv_ref[...] + b_ref[...]
    m = jnp.max(v, axis=1, keepdims=True)
    e = jnp.exp(v - m)
    o_ref[...] = e / jnp.sum(e, axis=1, keepdims=True)


def _softmax(acc, dinv, b):
    return pl.pallas_call(
        _softmax_body,
        grid=(MM_GRID,),
        in_specs=[
            pl.BlockSpec((NC, MM_ROWS, COUT), lambda i: (0, i, 0)),
            pl.BlockSpec((MM_ROWS, COUT), lambda i: (i, 0)),
            pl.BlockSpec((COUT,), lambda i: (0,)),
        ],
        out_specs=pl.BlockSpec((MM_ROWS, COUT), lambda i: (i, 0)),
        out_shape=jax.ShapeDtypeStruct((N_PAD, COUT), jnp.float32),
    )(acc, dinv, b)


def _sc_deg_body(dst_hbm, zero_hbm, deg_hbm, deg_s, dst_v, ones_v, sem):
    c = lax.axis_index("c")
    s = lax.axis_index("s")
    w = c * NS + s
    base = s * ROWS_PER_TILE

    def _ones(i, cy):
        ones_v[pl.ds(i * COUT, COUT)] = jnp.ones((COUT,), jnp.float32)
        return cy
    lax.fori_loop(0, EDGE_CHUNK // COUT, _ones, 0)

    pltpu.sync_copy(zero_hbm, deg_s.at[pl.ds(base, ROWS_PER_TILE)])
    pltpu.sync_copy(dst_hbm.at[w], dst_v)
    plsc.subcore_barrier()

    def _deg(j, cy):
        pltpu.async_copy(ones_v, deg_s.at[dst_v.at[j]], sem, add=True)
        return cy
    lax.fori_loop(0, CHUNKS_PER_WORKER, _deg, 0)

    def _drain(j, cy):
        pltpu.make_async_copy(ones_v, deg_s.at[dst_v.at[0]], sem).wait()
        return cy
    lax.fori_loop(0, CHUNKS_PER_WORKER, _drain, 0)
    plsc.subcore_barrier()

    pltpu.sync_copy(deg_s.at[pl.ds(base, ROWS_PER_TILE)],
                    deg_hbm.at[c, pl.ds(base, ROWS_PER_TILE)])


def _sc_msg_body(yw_hbm, zrows_hbm, src_hbm, dst_hbm, out_hbm,
                 ywt_s, acc_s, src_v, dst_v, rows_v, gsem, ssem):
    c = lax.axis_index("c")
    s = lax.axis_index("s")
    w = c * NS + s
    base = s * ROWS_PER_TILE

    pltpu.sync_copy(yw_hbm.at[pl.ds(base, ROWS_PER_TILE)],
                    ywt_s.at[pl.ds(base, ROWS_PER_TILE)])

    @pl.when(c == 0)
    def _():
        pltpu.sync_copy(yw_hbm.at[pl.ds(base, ROWS_PER_TILE)],
                        acc_s.at[pl.ds(base, ROWS_PER_TILE)])

    @pl.when(c != 0)
    def _():
        pltpu.sync_copy(zrows_hbm,
                        acc_s.at[pl.ds(base, ROWS_PER_TILE)])

    pltpu.sync_copy(src_hbm.at[w], src_v)
    pltpu.sync_copy(dst_hbm.at[w], dst_v)
    plsc.subcore_barrier()

    for p in range(DEPTH):
        pltpu.async_copy(ywt_s.at[src_v.at[p]], rows_v.at[p], gsem)

    def _msg(j, cy):
        pltpu.make_async_copy(ywt_s.at[src_v.at[j]], rows_v.at[j % RING],
                              gsem).wait()
        pltpu.async_copy(rows_v.at[j % RING], acc_s.at[dst_v.at[j]],
                         ssem, add=True)

        @pl.when(j >= DEPTH - 1)
        def _():
            pltpu.make_async_copy(rows_v.at[0], acc_s.at[dst_v.at[0]],
                                  ssem).wait()

        @pl.when(j + DEPTH < CHUNKS_PER_WORKER)
        def _():
            pltpu.async_copy(ywt_s.at[src_v.at[j + DEPTH]],
                             rows_v.at[(j + DEPTH) % RING], gsem)
        return cy
    lax.fori_loop(0, CHUNKS_PER_WORKER, _msg, 0)

    def _sdrain(j, cy):
        pltpu.make_async_copy(rows_v.at[0], acc_s.at[dst_v.at[0]],
                              ssem).wait()
        return cy
    lax.fori_loop(0, DEPTH - 1, _sdrain, 0)
    plsc.subcore_barrier()

    pltpu.sync_copy(acc_s.at[pl.ds(base, ROWS_PER_TILE)],
                    out_hbm.at[c, pl.ds(base, ROWS_PER_TILE)])


@jax.jit
def kernel(x, edge_index, W, b):
    n = x.shape[0]
    x_pad = jnp.zeros((N_PAD, DIM), jnp.float32).at[:n].set(x)
    zero_rows = jnp.zeros((ROWS_PER_TILE, COUT), jnp.float32)
    zero_1d = jnp.zeros((ROWS_PER_TILE,), jnp.float32)

    src = jnp.full((E_PAD,), n, jnp.int32).at[:N_EDGES].set(edge_index[0])
    dst = jnp.full((E_PAD,), n, jnp.int32).at[:N_EDGES].set(edge_index[1])
    src = src.reshape(NW, CHUNKS_PER_WORKER, EDGE_CHUNK)
    dst = dst.reshape(NW, CHUNKS_PER_WORKER, EDGE_CHUNK)

    mesh = plsc.VectorSubcoreMesh(core_axis_name="c", subcore_axis_name="s",
                                  num_cores=NC)
    sc_params = pltpu.CompilerParams(use_tc_tiling_on_sc=False)

    deg_kernel = pl.kernel(
        _sc_deg_body,
        out_type=jax.ShapeDtypeStruct((NC, N_PAD), jnp.float32),
        mesh=mesh,
        compiler_params=sc_params,
        scratch_types=[
            pltpu.VMEM_SHARED((N_PAD,), jnp.float32),
            pltpu.VMEM((CHUNKS_PER_WORKER, EDGE_CHUNK), jnp.int32),
            pltpu.VMEM((EDGE_CHUNK,), jnp.float32),
            pltpu.SemaphoreType.DMA,
        ],
    )
    deg = deg_kernel(dst, zero_1d)

    yw, dinv = _matmul_normalize(x_pad, W, deg)

    msg_kernel = pl.kernel(
        _sc_msg_body,
        out_type=jax.ShapeDtypeStruct((NC, N_PAD, COUT), jnp.float32),
        mesh=mesh,
        compiler_params=sc_params,
        scratch_types=[
            pltpu.VMEM_SHARED((N_PAD, COUT), jnp.float32),
            pltpu.VMEM_SHARED((N_PAD, COUT), jnp.float32),
            pltpu.VMEM((CHUNKS_PER_WORKER, EDGE_CHUNK), jnp.int32),
            pltpu.VMEM((CHUNKS_PER_WORKER, EDGE_CHUNK), jnp.int32),
            pltpu.VMEM((RING, EDGE_CHUNK, COUT), jnp.float32),
            pltpu.SemaphoreType.DMA,
            pltpu.SemaphoreType.DMA,
        ],
    )
    acc = msg_kernel(yw, zero_rows, src, dst)
    out = _softmax(acc, dinv, b)
    return out[:n]
